# Initial kernel scaffold; baseline (speedup 1.0000x reference)
import jax
import jax.numpy as jnp
from jax.experimental import pallas as pl


def kernel(X, edge_index, edge_weight, attention, W_conv_z, b_conv_z, W_lin_z, b_lin_z, W_conv_r, b_conv_r, W_lin_r, b_lin_r, W_conv_h, b_conv_h, W_lin_h, b_lin_h):
    raise NotImplementedError("write your pallas kernel here")


# speedup vs baseline: 25.9404x; 25.9404x over previous
import jax
import jax.numpy as jnp
from jax import lax
from jax.experimental import pallas as pl
from jax.experimental.pallas import tpu as pltpu
from jax.experimental.pallas import tpu_sc as plsc

N = 10000
E = 320000
F = 128
HD = 32
P = 12
GW = 2 * HD
RW = 2 * GW
U = P // 2
NPAD = 10240
EB = 80
NC = 2
NS = 16

_mesh = plsc.VectorSubcoreMesh(core_axis_name="c", subcore_axis_name="s")
_f32 = jnp.float32


def _wprep_body(wcz, wlz, bcz, blz, wch, wlh, bch, blh, att,
                wcat_o, bcat_o, probs_o):
    wlz_t = wlz[...][:HD, :]
    wlh_t = wlh[...][:HD, :]
    wz = jnp.dot(wcz[...], wlz_t, preferred_element_type=_f32)
    wh = jnp.dot(wch[...], wlh_t, preferred_element_type=_f32)
    wcat_o[...] = jnp.concatenate([wz, wh], axis=1)
    bz = jnp.dot(bcz[...], wlz_t, preferred_element_type=_f32) + blz[...]
    bh = jnp.dot(bch[...], wlh_t, preferred_element_type=_f32) + blh[...]
    bcat_o[...] = jnp.concatenate([bz, bh, bz, bh], axis=1)
    a = att[...]
    e = jnp.exp(a - jnp.max(a, axis=1, keepdims=True))
    probs_o[...] = e / jnp.sum(e, axis=1, keepdims=True)


def _wprep(wcz, wlz, bcz, blz, wch, wlh, bch, blh, att):
    return pl.pallas_call(
        _wprep_body,
        out_shape=[
            jax.ShapeDtypeStruct((F, GW), _f32),
            jax.ShapeDtypeStruct((1, RW), _f32),
            jax.ShapeDtypeStruct((1, P), _f32),
        ],
    )(wcz, wlz, bcz, blz, wch, wlh, bch, blh, att)


def _deg_body(dst_h, w_h, deg_o, dstv, wv, zbuf, acc_s):
    cid = lax.axis_index("c")
    sid = lax.axis_index("s")
    for i in range(NPAD // NS // 16):
        zbuf[pl.ds(i * 16, 16)] = jnp.zeros((16,), _f32)
    pltpu.sync_copy(zbuf, acc_s.at[pl.ds(sid * 640, 640)])
    plsc.subcore_barrier()
    ebase = cid * (E // NC) + sid * (E // (NC * NS))

    def body(b, carry):
        base = ebase + b * EB
        pltpu.sync_copy(dst_h.at[pl.ds(base, EB)], dstv)
        pltpu.sync_copy(w_h.at[pl.ds(base, EB)], wv)
        pltpu.sync_copy(wv, acc_s.at[dstv], add=True)
        return carry

    lax.fori_loop(0, E // (NC * NS) // EB, body, 0)
    plsc.subcore_barrier()
    pltpu.sync_copy(acc_s.at[pl.ds(sid * 640, 640)],
                    deg_o.at[cid, pl.ds(sid * 640, 640)])


def _deg(dst, w):
    return pl.kernel(
        _deg_body,
        out_type=jax.ShapeDtypeStruct((NC, NPAD), _f32),
        mesh=_mesh,
        scratch_types=[
            pltpu.VMEM((EB,), jnp.int32),
            pltpu.VMEM((EB,), _f32),
            pltpu.VMEM((NPAD // NS,), _f32),
            pltpu.VMEM_SHARED((NPAD,), _f32),
        ],
    )(dst, w)


def _gmat_body(xp, degt, wcat, gp_o, dinv_o):
    d = degt[:, 0:1] + degt[:, 1:2] + 1.0
    di = lax.rsqrt(d)
    g0 = jnp.dot(xp[0], wcat[...], preferred_element_type=_f32)
    g1 = jnp.dot(xp[1], wcat[...], preferred_element_type=_f32)
    gp_o[0] = jnp.concatenate([g0, g1], axis=1) * di
    dinv_o[...] = di


def _gmat(xp, degt, wcat, nb):
    bn = N // nb
    return pl.pallas_call(
        _gmat_body,
        grid=(U, nb),
        in_specs=[
            pl.BlockSpec((2, bn, F), lambda u, n: (u, n, 0)),
            pl.BlockSpec((bn, 2), lambda u, n: (n, 0)),
            pl.BlockSpec((F, GW), lambda u, n: (0, 0)),
        ],
        out_specs=[
            pl.BlockSpec((1, bn, RW), lambda u, n: (u, n, 0)),
            pl.BlockSpec((bn, 1), lambda u, n: (n, 0)),
        ],
        out_shape=[
            jax.ShapeDtypeStruct((U, N, RW), _f32),
            jax.ShapeDtypeStruct((N, 1), _f32),
        ],
    )(xp, degt, wcat)


def _prop_body(gflat, src_h, dst_h, w_h, acc_o,
               srcv, dstv, wv, abuf, rbuf, acc_s, semg):
    cid = lax.axis_index("c")
    sid = lax.axis_index("s")
    ept = E // NS
    for cc in range(U // NC):
        u = cid * (U // NC) + cc
        off = u * N

        @pl.when(sid == 0)
        def _():
            pltpu.sync_copy(gflat.at[pl.ds(off, N)], acc_s)

        plsc.subcore_barrier()
        ebase = sid * ept

        def body(b, carry):
            base = ebase + b * EB
            pltpu.sync_copy(src_h.at[pl.ds(base, EB)], srcv)
            pltpu.sync_copy(dst_h.at[pl.ds(base, EB)], dstv)
            pltpu.sync_copy(w_h.at[pl.ds(base, EB)], wv)
            for j in range(EB // 16):
                abuf[pl.ds(j * 16, 16)] = srcv[pl.ds(j * 16, 16)] + off
            pltpu.async_copy(gflat.at[abuf], rbuf, semg).wait()

            def jbody(j, jc):
                w16 = wv[pl.ds(j * 16, 16)]
                for i in range(16):
                    wi = lax.gather(
                        w16, jnp.full((16, 1), i, jnp.int32),
                        lax.GatherDimensionNumbers(
                            offset_dims=(), collapsed_slice_dims=(0,),
                            start_index_map=(0,)),
                        (1,), mode=lax.GatherScatterMode.PROMISE_IN_BOUNDS)
                    row = j * 16 + i
                    for k in range(RW // 16):
                        sl = pl.ds(k * 16, 16)
                        rbuf[row, sl] = rbuf[row, sl] * wi
                return jc

            lax.fori_loop(0, EB // 16, jbody, 0)
            pltpu.sync_copy(rbuf, acc_s.at[dstv], add=True)
            return carry

        lax.fori_loop(0, ept // EB, body, 0)
        plsc.subcore_barrier()

        @pl.when(sid <= NS - 2)
        def _():
            pltpu.sync_copy(acc_s.at[pl.ds(sid * 640, 640)],
                            acc_o.at[u, pl.ds(sid * 640, 640)])

        @pl.when(sid == NS - 1)
        def _():
            pltpu.sync_copy(
                acc_s.at[pl.ds((NS - 1) * 640, N - (NS - 1) * 640)],
                acc_o.at[u, pl.ds((NS - 1) * 640, N - (NS - 1) * 640)])

        plsc.subcore_barrier()


def _prop(gflat, src, dst, w):
    return pl.kernel(
        _prop_body,
        out_type=jax.ShapeDtypeStruct((U, N, RW), _f32),
        mesh=_mesh,
        scratch_types=[
            pltpu.VMEM((EB,), jnp.int32),
            pltpu.VMEM((EB,), jnp.int32),
            pltpu.VMEM((EB,), _f32),
            pltpu.VMEM((EB,), jnp.int32),
            pltpu.VMEM((EB, RW), _f32),
            pltpu.VMEM_SHARED((N, RW), _f32),
            pltpu.SemaphoreType.DMA,
        ],
    )(gflat, src, dst, w)


def _fin_body(acc, dinv, bcat, probs, out_o):
    u = pl.program_id(0)
    nb = pl.program_id(1)
    a = acc[0] * dinv[...] + bcat[...]
    z0 = jax.nn.sigmoid(a[:, 0:HD])
    t0 = jnp.tanh(a[:, HD:GW])
    z1 = jax.nn.sigmoid(a[:, GW:GW + HD])
    t1 = jnp.tanh(a[:, GW + HD:RW])
    pr0 = probs[0, 2 * u] * (1.0 / N)
    pr1 = probs[0, 2 * u + 1] * (1.0 / N)
    r = (pr0 * jnp.sum((1.0 - z0) * t0, axis=0, keepdims=True)
         + pr1 * jnp.sum((1.0 - z1) * t1, axis=0, keepdims=True))

    @pl.when((u == 0) & (nb == 0))
    def _():
        out_o[...] = r

    @pl.when((u > 0) | (nb > 0))
    def _():
        out_o[...] = out_o[...] + r


def _fin(acc, dinv, bcat, probs, nb):
    bn = N // nb
    return pl.pallas_call(
        _fin_body,
        grid=(U, nb),
        in_specs=[
            pl.BlockSpec((1, bn, RW), lambda u, n: (u, n, 0)),
            pl.BlockSpec((bn, 1), lambda u, n: (n, 0)),
            pl.BlockSpec((1, RW), lambda u, n: (0, 0)),
            pl.BlockSpec(memory_space=pltpu.SMEM),
        ],
        out_specs=pl.BlockSpec((1, HD), lambda u, n: (0, 0)),
        out_shape=jax.ShapeDtypeStruct((1, HD), _f32),
    )(acc, dinv, bcat, probs)


def kernel(X, edge_index, edge_weight, attention,
           W_conv_z, b_conv_z, W_lin_z, b_lin_z,
           W_conv_r, b_conv_r, W_lin_r, b_lin_r,
           W_conv_h, b_conv_h, W_lin_h, b_lin_h):
    ei = edge_index.astype(jnp.int32)
    src = ei[0]
    dst = ei[1]
    w = edge_weight.astype(_f32)

    wcat, bcat, probs = _wprep(
        W_conv_z, W_lin_z, b_conv_z.reshape(1, HD), b_lin_z.reshape(1, HD),
        W_conv_h, W_lin_h, b_conv_h.reshape(1, HD), b_lin_h.reshape(1, HD),
        attention.reshape(1, P))

    deg2 = _deg(dst, w)
    degt = deg2[:, :N].T

    xp = jnp.transpose(X, (2, 0, 1))
    gp, dinv = _gmat(xp, degt, wcat, nb=5)

    accf = _prop(gp.reshape(U * N, RW), src, dst, w)

    return _fin(accf, dinv, bcat, probs, nb=5)


# speedup vs baseline: 46.2221x; 1.7819x over previous
import jax
import jax.numpy as jnp
from jax import lax
from jax.experimental import pallas as pl
from jax.experimental.pallas import tpu as pltpu
from jax.experimental.pallas import tpu_sc as plsc

N = 10000
E = 320000
F = 128
HD = 32
P = 12
GW = 2 * HD
NPAD = 10240
EB = 80
NC = 2
NS = 16

_mesh = plsc.VectorSubcoreMesh(core_axis_name="c", subcore_axis_name="s")
_f32 = jnp.float32


def _wprep_body(wcz, wlz, bcz, blz, wch, wlh, bch, blh, att,
                wcat_o, bcat_o, probs_o):
    wlz_t = wlz[...][:HD, :]
    wlh_t = wlh[...][:HD, :]
    wz = jnp.dot(wcz[...], wlz_t, preferred_element_type=_f32)
    wh = jnp.dot(wch[...], wlh_t, preferred_element_type=_f32)
    wcat_o[...] = jnp.concatenate([wz, wh], axis=1)
    bz = jnp.dot(bcz[...], wlz_t, preferred_element_type=_f32) + blz[...]
    bh = jnp.dot(bch[...], wlh_t, preferred_element_type=_f32) + blh[...]
    bcat_o[...] = jnp.concatenate([bz, bh], axis=1)
    a = att[...]
    e = jnp.exp(a - jnp.max(a, axis=1, keepdims=True))
    probs_o[...] = e / jnp.sum(e, axis=1, keepdims=True)


def _wprep(wcz, wlz, bcz, blz, wch, wlh, bch, blh, att):
    return pl.pallas_call(
        _wprep_body,
        out_shape=[
            jax.ShapeDtypeStruct((F, GW), _f32),
            jax.ShapeDtypeStruct((1, GW), _f32),
            jax.ShapeDtypeStruct((1, P), _f32),
        ],
    )(wcz, wlz, bcz, blz, wch, wlh, bch, blh, att)


def _deg_body(dst_h, w_h, deg_o, dstb, wb, d0, d1, zbuf, acc_s, sems0, sems1):
    cid = lax.axis_index("c")
    sid = lax.axis_index("s")
    for i in range(NPAD // NS // 16):
        zbuf[pl.ds(i * 16, 16)] = jnp.zeros((16,), _f32)
    pltpu.sync_copy(zbuf, acc_s.at[pl.ds(sid * 640, 640)])
    ept = E // (NC * NS)
    nb = ept // EB
    eb0 = (cid * NS + sid) * ept
    pltpu.sync_copy(dst_h.at[pl.ds(eb0, ept)], dstb)
    pltpu.sync_copy(w_h.at[pl.ds(eb0, ept)], wb)
    dvs = (d0, d1)
    semss = (sems0, sems1)
    plsc.subcore_barrier()

    def body(t, carry):
        for r in range(2):
            b = t * 2 + r

            @pl.when(b >= 1)
            def _():
                pltpu.make_async_copy(
                    wb.at[pl.ds(0, EB)],
                    acc_s.at[dvs[1 - r]], semss[1 - r]).wait()

            for j in range(EB // 16):
                dvs[r][pl.ds(j * 16, 16)] = dstb[pl.ds(b * EB + j * 16, 16)]
            pltpu.async_copy(
                wb.at[pl.ds(b * EB, EB)], acc_s.at[dvs[r]], semss[r],
                add=True)
        return carry

    lax.fori_loop(0, nb // 2, body, 0)
    pltpu.make_async_copy(
        wb.at[pl.ds(0, EB)], acc_s.at[dvs[1]], semss[1]).wait()
    if nb % 2 == 1:
        b = nb - 1
        for j in range(EB // 16):
            dvs[0][pl.ds(j * 16, 16)] = dstb[pl.ds(b * EB + j * 16, 16)]
        pltpu.async_copy(
            wb.at[pl.ds(b * EB, EB)], acc_s.at[dvs[0]], semss[0], add=True)
        pltpu.make_async_copy(
            wb.at[pl.ds(0, EB)], acc_s.at[dvs[0]], semss[0]).wait()
    plsc.subcore_barrier()
    pltpu.sync_copy(acc_s.at[pl.ds(sid * 640, 640)],
                    deg_o.at[cid, pl.ds(sid * 640, 640)])


def _deg(dst, w):
    ept = E // (NC * NS)
    return pl.kernel(
        _deg_body,
        out_type=jax.ShapeDtypeStruct((NC, NPAD), _f32),
        mesh=_mesh,
        scratch_types=[
            pltpu.VMEM((ept,), jnp.int32),
            pltpu.VMEM((ept,), _f32),
            pltpu.VMEM((EB,), jnp.int32),
            pltpu.VMEM((EB,), jnp.int32),
            pltpu.VMEM((NPAD // NS,), _f32),
            pltpu.VMEM_SHARED((NPAD,), _f32),
            pltpu.SemaphoreType.DMA,
            pltpu.SemaphoreType.DMA,
        ],
    )(dst, w)


def _gmat_body(xp, degt, wcat, gp_o, dinv_o):
    d = degt[:, 0:1] + degt[:, 1:2] + 1.0
    di = lax.rsqrt(d)
    g = jnp.dot(xp[0], wcat[...], preferred_element_type=_f32)
    gp_o[0] = g * di
    dinv_o[...] = di


def _gmat(xp, degt, wcat, nb):
    bn = N // nb
    return pl.pallas_call(
        _gmat_body,
        grid=(P, nb),
        in_specs=[
            pl.BlockSpec((1, bn, F), lambda u, n: (u, n, 0)),
            pl.BlockSpec((bn, 2), lambda u, n: (n, 0)),
            pl.BlockSpec((F, GW), lambda u, n: (0, 0)),
        ],
        out_specs=[
            pl.BlockSpec((1, bn, GW), lambda u, n: (u, n, 0)),
            pl.BlockSpec((bn, 1), lambda u, n: (n, 0)),
        ],
        out_shape=[
            jax.ShapeDtypeStruct((P, N, GW), _f32),
            jax.ShapeDtypeStruct((N, 1), _f32),
        ],
    )(xp, degt, wcat)


def _prop_body(gflat, src_h, dst_h, w_h, acc_o,
               srcb, dstb, wb, a0, a1, d0, d1, r0, r1, acc_s,
               semg0, semg1, sems0, sems1):
    cid = lax.axis_index("c")
    sid = lax.axis_index("s")
    ept = E // NS
    nb = ept // EB
    eb0 = sid * ept
    pltpu.sync_copy(src_h.at[pl.ds(eb0, ept)], srcb)
    pltpu.sync_copy(dst_h.at[pl.ds(eb0, ept)], dstb)
    pltpu.sync_copy(w_h.at[pl.ds(eb0, ept)], wb)
    abufs = (a0, a1)
    dvs = (d0, d1)
    rbufs = (r0, r1)
    semgs = (semg0, semg1)
    semss = (sems0, sems1)

    for cc in range(P // NC):
        u = cid * (P // NC) + cc
        off = u * N

        @pl.when(sid == 0)
        def _():
            pltpu.sync_copy(gflat.at[pl.ds(off, N)], acc_s)

        plsc.subcore_barrier()

        for j in range(EB // 16):
            a0[pl.ds(j * 16, 16)] = srcb[pl.ds(j * 16, 16)] + off
        pltpu.async_copy(gflat.at[a0], r0, semg0)

        def body(t, carry):
            for r in range(2):
                b = t * 2 + r

                @pl.when(b >= 1)
                def _():
                    pltpu.make_async_copy(
                        rbufs[1 - r], acc_s.at[dvs[1 - r]], semss[1 - r]).wait()

                @pl.when(b + 1 < nb)
                def _():
                    for j in range(EB // 16):
                        sl = pl.ds(j * 16, 16)
                        abufs[1 - r][sl] = (
                            srcb[pl.ds((b + 1) * EB + j * 16, 16)] + off)
                    pltpu.async_copy(
                        gflat.at[abufs[1 - r]], rbufs[1 - r], semgs[1 - r])

                pltpu.make_async_copy(
                    gflat.at[abufs[r]], rbufs[r], semgs[r]).wait()

                def jbody(j, jc):
                    w16 = wb[pl.ds(b * EB + j * 16, 16)]
                    for i in range(16):
                        wi = lax.gather(
                            w16, jnp.full((16, 1), i, jnp.int32),
                            lax.GatherDimensionNumbers(
                                offset_dims=(), collapsed_slice_dims=(0,),
                                start_index_map=(0,)),
                            (1,),
                            mode=lax.GatherScatterMode.PROMISE_IN_BOUNDS)
                        row = j * 16 + i
                        for k in range(GW // 16):
                            sl = pl.ds(k * 16, 16)
                            rbufs[r][row, sl] = rbufs[r][row, sl] * wi
                    return jc

                lax.fori_loop(0, EB // 16, jbody, 0)
                for j in range(EB // 16):
                    dvs[r][pl.ds(j * 16, 16)] = (
                        dstb[pl.ds(b * EB + j * 16, 16)])
                pltpu.async_copy(
                    rbufs[r], acc_s.at[dvs[r]], semss[r], add=True)
            return carry

        lax.fori_loop(0, nb // 2, body, 0)
        pltpu.make_async_copy(
            rbufs[1], acc_s.at[dvs[1]], semss[1]).wait()
        plsc.subcore_barrier()

        @pl.when(sid <= NS - 2)
        def _():
            pltpu.sync_copy(acc_s.at[pl.ds(sid * 640, 640)],
                            acc_o.at[u, pl.ds(sid * 640, 640)])

        @pl.when(sid == NS - 1)
        def _():
            pltpu.sync_copy(
                acc_s.at[pl.ds((NS - 1) * 640, N - (NS - 1) * 640)],
                acc_o.at[u, pl.ds((NS - 1) * 640, N - (NS - 1) * 640)])

        plsc.subcore_barrier()


def _prop(gflat, src, dst, w):
    ept = E // NS
    return pl.kernel(
        _prop_body,
        out_type=jax.ShapeDtypeStruct((P, N, GW), _f32),
        mesh=_mesh,
        compiler_params=pltpu.CompilerParams(use_tc_tiling_on_sc=False),
        scratch_types=[
            pltpu.VMEM((ept,), jnp.int32),
            pltpu.VMEM((ept,), jnp.int32),
            pltpu.VMEM((ept,), _f32),
            pltpu.VMEM((EB,), jnp.int32),
            pltpu.VMEM((EB,), jnp.int32),
            pltpu.VMEM((EB,), jnp.int32),
            pltpu.VMEM((EB,), jnp.int32),
            pltpu.VMEM((EB, GW), _f32),
            pltpu.VMEM((EB, GW), _f32),
            pltpu.VMEM_SHARED((N, GW), _f32),
            pltpu.SemaphoreType.DMA,
            pltpu.SemaphoreType.DMA,
            pltpu.SemaphoreType.DMA,
            pltpu.SemaphoreType.DMA,
        ],
    )(gflat, src, dst, w)


def _fin_body(acc, dinv, bcat, probs, out_o):
    u = pl.program_id(0)
    nb = pl.program_id(1)
    a = acc[0] * dinv[...] + bcat[...]
    z = jax.nn.sigmoid(a[:, 0:HD])
    t = jnp.tanh(a[:, HD:GW])
    pr = probs[0, u] * (1.0 / N)
    r = pr * jnp.sum((1.0 - z) * t, axis=0, keepdims=True)

    @pl.when((u == 0) & (nb == 0))
    def _():
        out_o[...] = r

    @pl.when((u > 0) | (nb > 0))
    def _():
        out_o[...] = out_o[...] + r


def _fin(acc, dinv, bcat, probs, nb):
    bn = N // nb
    return pl.pallas_call(
        _fin_body,
        grid=(P, nb),
        in_specs=[
            pl.BlockSpec((1, bn, GW), lambda u, n: (u, n, 0)),
            pl.BlockSpec((bn, 1), lambda u, n: (n, 0)),
            pl.BlockSpec((1, GW), lambda u, n: (0, 0)),
            pl.BlockSpec(memory_space=pltpu.SMEM),
        ],
        out_specs=pl.BlockSpec((1, HD), lambda u, n: (0, 0)),
        out_shape=jax.ShapeDtypeStruct((1, HD), _f32),
    )(acc, dinv, bcat, probs)


def kernel(X, edge_index, edge_weight, attention,
           W_conv_z, b_conv_z, W_lin_z, b_lin_z,
           W_conv_r, b_conv_r, W_lin_r, b_lin_r,
           W_conv_h, b_conv_h, W_lin_h, b_lin_h):
    ei = edge_index.astype(jnp.int32)
    src = ei[0]
    dst = ei[1]
    w = edge_weight.astype(_f32)

    wcat, bcat, probs = _wprep(
        W_conv_z, W_lin_z, b_conv_z.reshape(1, HD), b_lin_z.reshape(1, HD),
        W_conv_h, W_lin_h, b_conv_h.reshape(1, HD), b_lin_h.reshape(1, HD),
        attention.reshape(1, P))

    deg2 = _deg(dst, w)
    degt = deg2[:, :N].T

    xp = jnp.transpose(X, (2, 0, 1))
    gp, dinv = _gmat(xp, degt, wcat, nb=5)

    accf = _prop(gp.reshape(P * N, GW), src, dst, w)

    return _fin(accf, dinv, bcat, probs, nb=5)


# speedup vs baseline: 50.8246x; 1.0996x over previous
import jax
import jax.numpy as jnp
from jax import lax
from jax.experimental import pallas as pl
from jax.experimental.pallas import tpu as pltpu
from jax.experimental.pallas import tpu_sc as plsc

N = 10000
E = 320000
F = 128
HD = 32
P = 12
GW = 2 * HD
NPAD = 10240
EB = 80
NC = 2
NS = 16

_mesh = plsc.VectorSubcoreMesh(core_axis_name="c", subcore_axis_name="s")
_f32 = jnp.float32


def _wprep_body(wcz, wlz, bcz, blz, wch, wlh, bch, blh, att,
                wcat_o, bcat_o, probs_o):
    wlz_t = wlz[...][:HD, :]
    wlh_t = wlh[...][:HD, :]
    wz = jnp.dot(wcz[...], wlz_t, preferred_element_type=_f32)
    wh = jnp.dot(wch[...], wlh_t, preferred_element_type=_f32)
    wcat_o[...] = jnp.concatenate([wz, wh], axis=1)
    bz = jnp.dot(bcz[...], wlz_t, preferred_element_type=_f32) + blz[...]
    bh = jnp.dot(bch[...], wlh_t, preferred_element_type=_f32) + blh[...]
    bcat_o[...] = jnp.concatenate([bz, bh], axis=1)
    a = att[...]
    e = jnp.exp(a - jnp.max(a, axis=1, keepdims=True))
    probs_o[...] = e / jnp.sum(e, axis=1, keepdims=True)


def _wprep(wcz, wlz, bcz, blz, wch, wlh, bch, blh, att):
    return pl.pallas_call(
        _wprep_body,
        out_shape=[
            jax.ShapeDtypeStruct((F, GW), _f32),
            jax.ShapeDtypeStruct((1, GW), _f32),
            jax.ShapeDtypeStruct((1, P), _f32),
        ],
    )(wcz, wlz, bcz, blz, wch, wlh, bch, blh, att)


def _deg_body(dst_h, w_h, deg_o, dstb, wb, d0, d1, zbuf, acc_s, sems0, sems1):
    cid = lax.axis_index("c")
    sid = lax.axis_index("s")
    for i in range(NPAD // NS // 16):
        zbuf[pl.ds(i * 16, 16)] = jnp.zeros((16,), _f32)
    pltpu.sync_copy(zbuf, acc_s.at[pl.ds(sid * 640, 640)])
    ept = E // (NC * NS)
    nb = ept // EB
    eb0 = (cid * NS + sid) * ept
    pltpu.sync_copy(dst_h.at[pl.ds(eb0, ept)], dstb)
    pltpu.sync_copy(w_h.at[pl.ds(eb0, ept)], wb)
    dvs = (d0, d1)
    semss = (sems0, sems1)
    plsc.subcore_barrier()

    def body(t, carry):
        for r in range(2):
            b = t * 2 + r

            @pl.when(b >= 1)
            def _():
                pltpu.make_async_copy(
                    wb.at[pl.ds(0, EB)],
                    acc_s.at[dvs[1 - r]], semss[1 - r]).wait()

            for j in range(EB // 16):
                dvs[r][pl.ds(j * 16, 16)] = dstb[pl.ds(b * EB + j * 16, 16)]
            pltpu.async_copy(
                wb.at[pl.ds(b * EB, EB)], acc_s.at[dvs[r]], semss[r],
                add=True)
        return carry

    lax.fori_loop(0, nb // 2, body, 0)
    pltpu.make_async_copy(
        wb.at[pl.ds(0, EB)], acc_s.at[dvs[1]], semss[1]).wait()
    if nb % 2 == 1:
        b = nb - 1
        for j in range(EB // 16):
            dvs[0][pl.ds(j * 16, 16)] = dstb[pl.ds(b * EB + j * 16, 16)]
        pltpu.async_copy(
            wb.at[pl.ds(b * EB, EB)], acc_s.at[dvs[0]], semss[0], add=True)
        pltpu.make_async_copy(
            wb.at[pl.ds(0, EB)], acc_s.at[dvs[0]], semss[0]).wait()
    plsc.subcore_barrier()
    pltpu.sync_copy(acc_s.at[pl.ds(sid * 640, 640)],
                    deg_o.at[cid, pl.ds(sid * 640, 640)])


def _deg(dst, w):
    ept = E // (NC * NS)
    return pl.kernel(
        _deg_body,
        out_type=jax.ShapeDtypeStruct((NC, NPAD), _f32),
        mesh=_mesh,
        scratch_types=[
            pltpu.VMEM((ept,), jnp.int32),
            pltpu.VMEM((ept,), _f32),
            pltpu.VMEM((EB,), jnp.int32),
            pltpu.VMEM((EB,), jnp.int32),
            pltpu.VMEM((NPAD // NS,), _f32),
            pltpu.VMEM_SHARED((NPAD,), _f32),
            pltpu.SemaphoreType.DMA,
            pltpu.SemaphoreType.DMA,
        ],
    )(dst, w)


def _gmat_body(xp, degt, wcat, gp_o, dinv_o):
    d = degt[:, 0:1] + degt[:, 1:2] + 1.0
    di = lax.rsqrt(d)
    g = jnp.dot(xp[0], wcat[...], preferred_element_type=_f32)
    gp_o[0] = g * di
    dinv_o[...] = di


def _gmat(xp, degt, wcat, nb):
    bn = N // nb
    return pl.pallas_call(
        _gmat_body,
        grid=(P, nb),
        in_specs=[
            pl.BlockSpec((1, bn, F), lambda u, n: (u, n, 0)),
            pl.BlockSpec((bn, 2), lambda u, n: (n, 0)),
            pl.BlockSpec((F, GW), lambda u, n: (0, 0)),
        ],
        out_specs=[
            pl.BlockSpec((1, bn, GW), lambda u, n: (u, n, 0)),
            pl.BlockSpec((bn, 1), lambda u, n: (n, 0)),
        ],
        out_shape=[
            jax.ShapeDtypeStruct((P, N, GW), _f32),
            jax.ShapeDtypeStruct((N, 1), _f32),
        ],
    )(xp, degt, wcat)


def _prop_body(gflat, src_h, dst_h, w_h, acc_o,
               srcb, dstb, wb, a0, a1, d0, d1, r0, r1, acc_s,
               semg0, semg1, sems0, sems1):
    cid = lax.axis_index("c")
    sid = lax.axis_index("s")
    ept = E // NS
    nb = ept // EB
    eb0 = sid * ept
    pltpu.sync_copy(src_h.at[pl.ds(eb0, ept)], srcb)
    pltpu.sync_copy(dst_h.at[pl.ds(eb0, ept)], dstb)
    pltpu.sync_copy(w_h.at[pl.ds(eb0, ept)], wb)
    abufs = (a0, a1)
    dvs = (d0, d1)
    rbufs = (r0, r1)
    semgs = (semg0, semg1)
    semss = (sems0, sems1)

    def chunk_body(cc, ccarry):
        u = cid * (P // NC) + cc
        off = u * N

        @pl.when(sid == 0)
        def _():
            pltpu.sync_copy(gflat.at[pl.ds(off, N)], acc_s)

        plsc.subcore_barrier()

        for j in range(EB // 16):
            a0[pl.ds(j * 16, 16)] = srcb[pl.ds(j * 16, 16)] + off
        pltpu.async_copy(gflat.at[a0], r0, semg0)

        def body(t, carry):
            for r in range(2):
                b = t * 2 + r

                @pl.when(b >= 1)
                def _():
                    pltpu.make_async_copy(
                        rbufs[1 - r], acc_s.at[dvs[1 - r]], semss[1 - r]).wait()

                @pl.when(b + 1 < nb)
                def _():
                    for j in range(EB // 16):
                        sl = pl.ds(j * 16, 16)
                        abufs[1 - r][sl] = (
                            srcb[pl.ds((b + 1) * EB + j * 16, 16)] + off)
                    pltpu.async_copy(
                        gflat.at[abufs[1 - r]], rbufs[1 - r], semgs[1 - r])

                pltpu.make_async_copy(
                    gflat.at[abufs[r]], rbufs[r], semgs[r]).wait()

                for j in range(EB // 16):
                    w16 = wb[pl.ds(b * EB + j * 16, 16)]
                    for i in range(16):
                        wi = lax.gather(
                            w16, jnp.full((16, 1), i, jnp.int32),
                            lax.GatherDimensionNumbers(
                                offset_dims=(), collapsed_slice_dims=(0,),
                                start_index_map=(0,)),
                            (1,),
                            mode=lax.GatherScatterMode.PROMISE_IN_BOUNDS)
                        row = j * 16 + i
                        for k in range(GW // 16):
                            sl = pl.ds(k * 16, 16)
                            rbufs[r][row, sl] = rbufs[r][row, sl] * wi

                for j in range(EB // 16):
                    dvs[r][pl.ds(j * 16, 16)] = (
                        dstb[pl.ds(b * EB + j * 16, 16)])
                pltpu.async_copy(
                    rbufs[r], acc_s.at[dvs[r]], semss[r], add=True)
            return carry

        lax.fori_loop(0, nb // 2, body, 0)
        pltpu.make_async_copy(
            rbufs[1], acc_s.at[dvs[1]], semss[1]).wait()
        plsc.subcore_barrier()

        @pl.when(sid <= NS - 2)
        def _():
            pltpu.sync_copy(acc_s.at[pl.ds(sid * 640, 640)],
                            acc_o.at[u, pl.ds(sid * 640, 640)])

        @pl.when(sid == NS - 1)
        def _():
            pltpu.sync_copy(
                acc_s.at[pl.ds((NS - 1) * 640, N - (NS - 1) * 640)],
                acc_o.at[u, pl.ds((NS - 1) * 640, N - (NS - 1) * 640)])

        plsc.subcore_barrier()
        return ccarry

    lax.fori_loop(0, P // NC, chunk_body, 0)


def _prop(gflat, src, dst, w):
    ept = E // NS
    return pl.kernel(
        _prop_body,
        out_type=jax.ShapeDtypeStruct((P, N, GW), _f32),
        mesh=_mesh,
        compiler_params=pltpu.CompilerParams(use_tc_tiling_on_sc=False),
        scratch_types=[
            pltpu.VMEM((ept,), jnp.int32),
            pltpu.VMEM((ept,), jnp.int32),
            pltpu.VMEM((ept,), _f32),
            pltpu.VMEM((EB,), jnp.int32),
            pltpu.VMEM((EB,), jnp.int32),
            pltpu.VMEM((EB,), jnp.int32),
            pltpu.VMEM((EB,), jnp.int32),
            pltpu.VMEM((EB, GW), _f32),
            pltpu.VMEM((EB, GW), _f32),
            pltpu.VMEM_SHARED((N, GW), _f32),
            pltpu.SemaphoreType.DMA,
            pltpu.SemaphoreType.DMA,
            pltpu.SemaphoreType.DMA,
            pltpu.SemaphoreType.DMA,
        ],
    )(gflat, src, dst, w)


def _fin_body(acc, dinv, bcat, probs, out_o):
    u = pl.program_id(0)
    nb = pl.program_id(1)
    a = acc[0] * dinv[...] + bcat[...]
    z = jax.nn.sigmoid(a[:, 0:HD])
    t = jnp.tanh(a[:, HD:GW])
    pr = probs[0, u] * (1.0 / N)
    r = pr * jnp.sum((1.0 - z) * t, axis=0, keepdims=True)

    @pl.when((u == 0) & (nb == 0))
    def _():
        out_o[...] = r

    @pl.when((u > 0) | (nb > 0))
    def _():
        out_o[...] = out_o[...] + r


def _fin(acc, dinv, bcat, probs, nb):
    bn = N // nb
    return pl.pallas_call(
        _fin_body,
        grid=(P, nb),
        in_specs=[
            pl.BlockSpec((1, bn, GW), lambda u, n: (u, n, 0)),
            pl.BlockSpec((bn, 1), lambda u, n: (n, 0)),
            pl.BlockSpec((1, GW), lambda u, n: (0, 0)),
            pl.BlockSpec(memory_space=pltpu.SMEM),
        ],
        out_specs=pl.BlockSpec((1, HD), lambda u, n: (0, 0)),
        out_shape=jax.ShapeDtypeStruct((1, HD), _f32),
    )(acc, dinv, bcat, probs)


def kernel(X, edge_index, edge_weight, attention,
           W_conv_z, b_conv_z, W_lin_z, b_lin_z,
           W_conv_r, b_conv_r, W_lin_r, b_lin_r,
           W_conv_h, b_conv_h, W_lin_h, b_lin_h):
    ei = edge_index.astype(jnp.int32)
    src = ei[0]
    dst = ei[1]
    w = edge_weight.astype(_f32)

    wcat, bcat, probs = _wprep(
        W_conv_z, W_lin_z, b_conv_z.reshape(1, HD), b_lin_z.reshape(1, HD),
        W_conv_h, W_lin_h, b_conv_h.reshape(1, HD), b_lin_h.reshape(1, HD),
        attention.reshape(1, P))

    deg2 = _deg(dst, w)
    degt = deg2[:, :N].T

    xp = jnp.transpose(X, (2, 0, 1))
    gp, dinv = _gmat(xp, degt, wcat, nb=5)

    accf = _prop(gp.reshape(P * N, GW), src, dst, w)

    return _fin(accf, dinv, bcat, probs, nb=5)


# speedup vs baseline: 55.3419x; 1.0889x over previous
import jax
import jax.numpy as jnp
from jax import lax
from jax.experimental import pallas as pl
from jax.experimental.pallas import tpu as pltpu
from jax.experimental.pallas import tpu_sc as plsc

N = 10000
E = 320000
F = 128
HD = 32
P = 12
GW = 2 * HD
NPAD = 10240
EB = 80
NC = 2
NS = 16

_mesh = plsc.VectorSubcoreMesh(core_axis_name="c", subcore_axis_name="s")
_f32 = jnp.float32


def _wprep_body(wcz, wlz, bcz, blz, wch, wlh, bch, blh, att,
                wcat_o, bcat_o, probs_o):
    wlz_t = wlz[...][:HD, :]
    wlh_t = wlh[...][:HD, :]
    wz = jnp.dot(wcz[...], wlz_t, preferred_element_type=_f32)
    wh = jnp.dot(wch[...], wlh_t, preferred_element_type=_f32)
    wcat_o[...] = jnp.concatenate([wz, wh], axis=1)
    bz = jnp.dot(bcz[...], wlz_t, preferred_element_type=_f32) + blz[...]
    bh = jnp.dot(bch[...], wlh_t, preferred_element_type=_f32) + blh[...]
    bcat_o[...] = jnp.concatenate([bz, bh], axis=1)
    a = att[...]
    e = jnp.exp(a - jnp.max(a, axis=1, keepdims=True))
    probs_o[...] = e / jnp.sum(e, axis=1, keepdims=True)


def _wprep(wcz, wlz, bcz, blz, wch, wlh, bch, blh, att):
    return pl.pallas_call(
        _wprep_body,
        out_shape=[
            jax.ShapeDtypeStruct((F, GW), _f32),
            jax.ShapeDtypeStruct((1, GW), _f32),
            jax.ShapeDtypeStruct((1, P), _f32),
        ],
    )(wcz, wlz, bcz, blz, wch, wlh, bch, blh, att)


def _deg_body(dst_h, w_h, deg_o, dstb, wb, d0, d1, zbuf, acc_s, sems0, sems1):
    cid = lax.axis_index("c")
    sid = lax.axis_index("s")
    for i in range(NPAD // NS // 16):
        zbuf[pl.ds(i * 16, 16)] = jnp.zeros((16,), _f32)
    pltpu.sync_copy(zbuf, acc_s.at[pl.ds(sid * 640, 640)])
    ept = E // (NC * NS)
    nb = ept // EB
    eb0 = (cid * NS + sid) * ept
    pltpu.sync_copy(dst_h.at[pl.ds(eb0, ept)], dstb)
    pltpu.sync_copy(w_h.at[pl.ds(eb0, ept)], wb)
    dvs = (d0, d1)
    semss = (sems0, sems1)
    plsc.subcore_barrier()

    def body(t, carry):
        for r in range(2):
            b = t * 2 + r

            @pl.when(b >= 1)
            def _():
                pltpu.make_async_copy(
                    wb.at[pl.ds(0, EB)],
                    acc_s.at[dvs[1 - r]], semss[1 - r]).wait()

            for j in range(EB // 16):
                dvs[r][pl.ds(j * 16, 16)] = dstb[pl.ds(b * EB + j * 16, 16)]
            pltpu.async_copy(
                wb.at[pl.ds(b * EB, EB)], acc_s.at[dvs[r]], semss[r],
                add=True)
        return carry

    lax.fori_loop(0, nb // 2, body, 0)
    pltpu.make_async_copy(
        wb.at[pl.ds(0, EB)], acc_s.at[dvs[1]], semss[1]).wait()
    if nb % 2 == 1:
        b = nb - 1
        for j in range(EB // 16):
            dvs[0][pl.ds(j * 16, 16)] = dstb[pl.ds(b * EB + j * 16, 16)]
        pltpu.async_copy(
            wb.at[pl.ds(b * EB, EB)], acc_s.at[dvs[0]], semss[0], add=True)
        pltpu.make_async_copy(
            wb.at[pl.ds(0, EB)], acc_s.at[dvs[0]], semss[0]).wait()
    plsc.subcore_barrier()
    pltpu.sync_copy(acc_s.at[pl.ds(sid * 640, 640)],
                    deg_o.at[cid, pl.ds(sid * 640, 640)])


def _deg(dst, w):
    ept = E // (NC * NS)
    return pl.kernel(
        _deg_body,
        out_type=jax.ShapeDtypeStruct((NC, NPAD), _f32),
        mesh=_mesh,
        scratch_types=[
            pltpu.VMEM((ept,), jnp.int32),
            pltpu.VMEM((ept,), _f32),
            pltpu.VMEM((EB,), jnp.int32),
            pltpu.VMEM((EB,), jnp.int32),
            pltpu.VMEM((NPAD // NS,), _f32),
            pltpu.VMEM_SHARED((NPAD,), _f32),
            pltpu.SemaphoreType.DMA,
            pltpu.SemaphoreType.DMA,
        ],
    )(dst, w)


def _gmat_body(xp, degt, wcat, gp_o, dinv_o):
    d = degt[:, 0:1] + degt[:, 1:2] + 1.0
    di = lax.rsqrt(d)
    g = jnp.dot(xp[0], wcat[...], preferred_element_type=_f32)
    gp_o[0] = g * di
    dinv_o[...] = di


def _gmat(xp, degt, wcat, nb):
    bn = N // nb
    return pl.pallas_call(
        _gmat_body,
        grid=(P, nb),
        in_specs=[
            pl.BlockSpec((1, bn, F), lambda u, n: (u, n, 0)),
            pl.BlockSpec((bn, 2), lambda u, n: (n, 0)),
            pl.BlockSpec((F, GW), lambda u, n: (0, 0)),
        ],
        out_specs=[
            pl.BlockSpec((1, bn, GW), lambda u, n: (u, n, 0)),
            pl.BlockSpec((bn, 1), lambda u, n: (n, 0)),
        ],
        out_shape=[
            jax.ShapeDtypeStruct((P, N, GW), _f32),
            jax.ShapeDtypeStruct((N, 1), _f32),
        ],
    )(xp, degt, wcat)


def _prop_body(gflat, src_h, dst_h, w_h, acc_o,
               srcb, dstb, wb,
               a0, a1, a2, a3, a4, d0, d1, d2, d3, d4,
               r0, r1, r2, r3, r4, acc_s,
               semg0, semg1, semg2, semg3, semg4,
               sems0, sems1, sems2, sems3, sems4):
    cid = lax.axis_index("c")
    sid = lax.axis_index("s")
    ept = E // NS
    nb = ept // EB
    RD = 5
    eb0 = sid * ept
    pltpu.sync_copy(src_h.at[pl.ds(eb0, ept)], srcb)
    pltpu.sync_copy(dst_h.at[pl.ds(eb0, ept)], dstb)
    pltpu.sync_copy(w_h.at[pl.ds(eb0, ept)], wb)
    abufs = (a0, a1, a2, a3, a4)
    dvs = (d0, d1, d2, d3, d4)
    rbufs = (r0, r1, r2, r3, r4)
    semgs = (semg0, semg1, semg2, semg3, semg4)
    semss = (sems0, sems1, sems2, sems3, sems4)

    def _abs_fire(b, s, off):
        for j in range(EB // 16):
            sl = pl.ds(j * 16, 16)
            abufs[s][sl] = srcb[pl.ds(b * EB + j * 16, 16)] + off
        pltpu.async_copy(gflat.at[abufs[s]], rbufs[s], semgs[s])

    def chunk_body(cc, ccarry):
        u = cid * (P // NC) + cc
        off = u * N

        @pl.when(sid == 0)
        def _():
            pltpu.sync_copy(gflat.at[pl.ds(off, N)], acc_s)

        plsc.subcore_barrier()

        _abs_fire(0, 0, off)
        _abs_fire(1, 1, off)

        def body(t, carry):
            for r in range(RD):
                b = t * RD + r

                s3 = (r - 3) % RD
                s2 = (r + 2) % RD

                @pl.when(b >= 3)
                def _():
                    pltpu.make_async_copy(
                        rbufs[s3], acc_s.at[dvs[s3]], semss[s3]).wait()

                @pl.when(b + 2 < nb)
                def _():
                    _abs_fire(b + 2, s2, off)

                pltpu.make_async_copy(
                    gflat.at[abufs[r]], rbufs[r], semgs[r]).wait()

                for j in range(EB // 16):
                    w16 = wb[pl.ds(b * EB + j * 16, 16)]
                    for i in range(16):
                        wi = lax.gather(
                            w16, jnp.full((16, 1), i, jnp.int32),
                            lax.GatherDimensionNumbers(
                                offset_dims=(), collapsed_slice_dims=(0,),
                                start_index_map=(0,)),
                            (1,),
                            mode=lax.GatherScatterMode.PROMISE_IN_BOUNDS)
                        row = j * 16 + i
                        for k in range(GW // 16):
                            sl = pl.ds(k * 16, 16)
                            rbufs[r][row, sl] = rbufs[r][row, sl] * wi

                for j in range(EB // 16):
                    dvs[r][pl.ds(j * 16, 16)] = (
                        dstb[pl.ds(b * EB + j * 16, 16)])
                pltpu.async_copy(
                    rbufs[r], acc_s.at[dvs[r]], semss[r], add=True)
            return carry

        lax.fori_loop(0, nb // RD, body, 0)
        for s3 in ((nb - 3) % RD, (nb - 2) % RD, (nb - 1) % RD):
            pltpu.make_async_copy(
                rbufs[s3], acc_s.at[dvs[s3]], semss[s3]).wait()
        plsc.subcore_barrier()

        @pl.when(sid <= NS - 2)
        def _():
            pltpu.sync_copy(acc_s.at[pl.ds(sid * 640, 640)],
                            acc_o.at[u, pl.ds(sid * 640, 640)])

        @pl.when(sid == NS - 1)
        def _():
            pltpu.sync_copy(
                acc_s.at[pl.ds((NS - 1) * 640, N - (NS - 1) * 640)],
                acc_o.at[u, pl.ds((NS - 1) * 640, N - (NS - 1) * 640)])

        plsc.subcore_barrier()
        return ccarry

    lax.fori_loop(0, P // NC, chunk_body, 0)


def _prop(gflat, src, dst, w):
    ept = E // NS
    return pl.kernel(
        _prop_body,
        out_type=jax.ShapeDtypeStruct((P, N, GW), _f32),
        mesh=_mesh,
        compiler_params=pltpu.CompilerParams(use_tc_tiling_on_sc=False),
        scratch_types=(
            [pltpu.VMEM((ept,), jnp.int32),
             pltpu.VMEM((ept,), jnp.int32),
             pltpu.VMEM((ept,), _f32)]
            + [pltpu.VMEM((EB,), jnp.int32) for _ in range(10)]
            + [pltpu.VMEM((EB, GW), _f32) for _ in range(5)]
            + [pltpu.VMEM_SHARED((N, GW), _f32)]
            + [pltpu.SemaphoreType.DMA for _ in range(10)]
        ),
    )(gflat, src, dst, w)


def _fin_body(acc, dinv, bcat, probs, out_o):
    u = pl.program_id(0)
    nb = pl.program_id(1)
    a = acc[0] * dinv[...] + bcat[...]
    z = jax.nn.sigmoid(a[:, 0:HD])
    t = jnp.tanh(a[:, HD:GW])
    pr = probs[0, u] * (1.0 / N)
    r = pr * jnp.sum((1.0 - z) * t, axis=0, keepdims=True)

    @pl.when((u == 0) & (nb == 0))
    def _():
        out_o[...] = r

    @pl.when((u > 0) | (nb > 0))
    def _():
        out_o[...] = out_o[...] + r


def _fin(acc, dinv, bcat, probs, nb):
    bn = N // nb
    return pl.pallas_call(
        _fin_body,
        grid=(P, nb),
        in_specs=[
            pl.BlockSpec((1, bn, GW), lambda u, n: (u, n, 0)),
            pl.BlockSpec((bn, 1), lambda u, n: (n, 0)),
            pl.BlockSpec((1, GW), lambda u, n: (0, 0)),
            pl.BlockSpec(memory_space=pltpu.SMEM),
        ],
        out_specs=pl.BlockSpec((1, HD), lambda u, n: (0, 0)),
        out_shape=jax.ShapeDtypeStruct((1, HD), _f32),
    )(acc, dinv, bcat, probs)


def kernel(X, edge_index, edge_weight, attention,
           W_conv_z, b_conv_z, W_lin_z, b_lin_z,
           W_conv_r, b_conv_r, W_lin_r, b_lin_r,
           W_conv_h, b_conv_h, W_lin_h, b_lin_h):
    ei = edge_index.astype(jnp.int32)
    src = ei[0]
    dst = ei[1]
    w = edge_weight.astype(_f32)

    wcat, bcat, probs = _wprep(
        W_conv_z, W_lin_z, b_conv_z.reshape(1, HD), b_lin_z.reshape(1, HD),
        W_conv_h, W_lin_h, b_conv_h.reshape(1, HD), b_lin_h.reshape(1, HD),
        attention.reshape(1, P))

    deg2 = _deg(dst, w)
    degt = deg2[:, :N].T

    xp = jnp.transpose(X, (2, 0, 1))
    gp, dinv = _gmat(xp, degt, wcat, nb=5)

    accf = _prop(gp.reshape(P * N, GW), src, dst, w)

    return _fin(accf, dinv, bcat, probs, nb=5)


# speedup vs baseline: 56.9582x; 1.0292x over previous
import jax
import jax.numpy as jnp
from jax import lax
from jax.experimental import pallas as pl
from jax.experimental.pallas import tpu as pltpu
from jax.experimental.pallas import tpu_sc as plsc

N = 10000
E = 320000
F = 128
HD = 32
P = 12
GW = 2 * HD
NPAD = 10240
EB = 80
NC = 2
NS = 16

_mesh = plsc.VectorSubcoreMesh(core_axis_name="c", subcore_axis_name="s")
_f32 = jnp.float32


def _wprep_body(wcz, wlz, bcz, blz, wch, wlh, bch, blh, att,
                wcat_o, bcat_o, probs_o):
    wlz_t = wlz[...][:HD, :]
    wlh_t = wlh[...][:HD, :]
    wz = jnp.dot(wcz[...], wlz_t, preferred_element_type=_f32)
    wh = jnp.dot(wch[...], wlh_t, preferred_element_type=_f32)
    wcat_o[...] = jnp.concatenate([wz, wh], axis=1)
    bz = jnp.dot(bcz[...], wlz_t, preferred_element_type=_f32) + blz[...]
    bh = jnp.dot(bch[...], wlh_t, preferred_element_type=_f32) + blh[...]
    bcat_o[...] = jnp.concatenate([bz, bh], axis=1)
    a = att[...]
    e = jnp.exp(a - jnp.max(a, axis=1, keepdims=True))
    probs_o[...] = e / jnp.sum(e, axis=1, keepdims=True)


def _wprep(wcz, wlz, bcz, blz, wch, wlh, bch, blh, att):
    return pl.pallas_call(
        _wprep_body,
        out_shape=[
            jax.ShapeDtypeStruct((F, GW), _f32),
            jax.ShapeDtypeStruct((1, GW), _f32),
            jax.ShapeDtypeStruct((1, P), _f32),
        ],
    )(wcz, wlz, bcz, blz, wch, wlh, bch, blh, att)


def _deg_body(dst_h, w_h, deg_o, dstb, wb, d0, d1, zbuf, acc_s, sems0, sems1):
    cid = lax.axis_index("c")
    sid = lax.axis_index("s")
    for i in range(NPAD // NS // 16):
        zbuf[pl.ds(i * 16, 16)] = jnp.zeros((16,), _f32)
    pltpu.sync_copy(zbuf, acc_s.at[pl.ds(sid * 640, 640)])
    ept = E // (NC * NS)
    nb = ept // EB
    eb0 = (cid * NS + sid) * ept
    pltpu.sync_copy(dst_h.at[pl.ds(eb0, ept)], dstb)
    pltpu.sync_copy(w_h.at[pl.ds(eb0, ept)], wb)
    dvs = (d0, d1)
    semss = (sems0, sems1)
    plsc.subcore_barrier()

    def body(t, carry):
        for r in range(2):
            b = t * 2 + r

            @pl.when(b >= 1)
            def _():
                pltpu.make_async_copy(
                    wb.at[pl.ds(0, EB)],
                    acc_s.at[dvs[1 - r]], semss[1 - r]).wait()

            for j in range(EB // 16):
                dvs[r][pl.ds(j * 16, 16)] = dstb[pl.ds(b * EB + j * 16, 16)]
            pltpu.async_copy(
                wb.at[pl.ds(b * EB, EB)], acc_s.at[dvs[r]], semss[r],
                add=True)
        return carry

    lax.fori_loop(0, nb // 2, body, 0)
    pltpu.make_async_copy(
        wb.at[pl.ds(0, EB)], acc_s.at[dvs[1]], semss[1]).wait()
    if nb % 2 == 1:
        b = nb - 1
        for j in range(EB // 16):
            dvs[0][pl.ds(j * 16, 16)] = dstb[pl.ds(b * EB + j * 16, 16)]
        pltpu.async_copy(
            wb.at[pl.ds(b * EB, EB)], acc_s.at[dvs[0]], semss[0], add=True)
        pltpu.make_async_copy(
            wb.at[pl.ds(0, EB)], acc_s.at[dvs[0]], semss[0]).wait()
    plsc.subcore_barrier()
    pltpu.sync_copy(acc_s.at[pl.ds(sid * 640, 640)],
                    deg_o.at[cid, pl.ds(sid * 640, 640)])


def _deg(dst, w):
    ept = E // (NC * NS)
    return pl.kernel(
        _deg_body,
        out_type=jax.ShapeDtypeStruct((NC, NPAD), _f32),
        mesh=_mesh,
        scratch_types=[
            pltpu.VMEM((ept,), jnp.int32),
            pltpu.VMEM((ept,), _f32),
            pltpu.VMEM((EB,), jnp.int32),
            pltpu.VMEM((EB,), jnp.int32),
            pltpu.VMEM((NPAD // NS,), _f32),
            pltpu.VMEM_SHARED((NPAD,), _f32),
            pltpu.SemaphoreType.DMA,
            pltpu.SemaphoreType.DMA,
        ],
    )(dst, w)


def _gmat_body(xp, degt, wcat, gp_o, dinv_o):
    d = degt[:, 0:1] + degt[:, 1:2] + 1.0
    di = lax.rsqrt(d)
    g = jnp.dot(xp[0], wcat[...], preferred_element_type=_f32)
    gp_o[0] = g * di
    dinv_o[...] = di


def _gmat(xp, degt, wcat, nb):
    bn = N // nb
    return pl.pallas_call(
        _gmat_body,
        grid=(P, nb),
        in_specs=[
            pl.BlockSpec((1, bn, F), lambda u, n: (u, n, 0)),
            pl.BlockSpec((bn, 2), lambda u, n: (n, 0)),
            pl.BlockSpec((F, GW), lambda u, n: (0, 0)),
        ],
        out_specs=[
            pl.BlockSpec((1, bn, GW), lambda u, n: (u, n, 0)),
            pl.BlockSpec((bn, 1), lambda u, n: (n, 0)),
        ],
        out_shape=[
            jax.ShapeDtypeStruct((P, N, GW), _f32),
            jax.ShapeDtypeStruct((N, 1), _f32),
        ],
    )(xp, degt, wcat)


def _prop_body(gflat, src_h, dst_h, w_h, acc_o,
               srcb, dstb, wb,
               a0, a1, a2, a3, a4, d0, d1, d2, d3, d4,
               r0, r1, r2, r3, r4, acc_s,
               semg0, semg1, semg2, semg3, semg4,
               sems0, sems1, sems2, sems3, sems4):
    cid = lax.axis_index("c")
    sid = lax.axis_index("s")
    ept = E // NS
    nb = ept // EB
    RD = 5
    eb0 = sid * ept
    pltpu.sync_copy(src_h.at[pl.ds(eb0, ept)], srcb)
    pltpu.sync_copy(dst_h.at[pl.ds(eb0, ept)], dstb)
    pltpu.sync_copy(w_h.at[pl.ds(eb0, ept)], wb)
    abufs = (a0, a1, a2, a3, a4)
    dvs = (d0, d1, d2, d3, d4)
    rbufs = (r0, r1, r2, r3, r4)
    semgs = (semg0, semg1, semg2, semg3, semg4)
    semss = (sems0, sems1, sems2, sems3, sems4)

    def _abs_fire(b, s, off):
        for j in range(EB // 16):
            sl = pl.ds(j * 16, 16)
            abufs[s][sl] = srcb[pl.ds(b * EB + j * 16, 16)] + off
        pltpu.async_copy(gflat.at[abufs[s]], rbufs[s], semgs[s])

    def chunk_body(cc, ccarry):
        u = cid * (P // NC) + cc
        off = u * N

        @pl.when(sid == 0)
        def _():
            pltpu.sync_copy(gflat.at[pl.ds(off, N)], acc_s)

        plsc.subcore_barrier()

        _abs_fire(0, 0, off)
        _abs_fire(1, 1, off)
        _abs_fire(2, 2, off)

        def body(t, carry):
            for r in range(RD):
                b = t * RD + r

                s3 = (r - 2) % RD
                s2 = (r + 3) % RD

                @pl.when(b >= 2)
                def _():
                    pltpu.make_async_copy(
                        rbufs[s3], acc_s.at[dvs[s3]], semss[s3]).wait()

                @pl.when(b + 3 < nb)
                def _():
                    _abs_fire(b + 3, s2, off)

                pltpu.make_async_copy(
                    gflat.at[abufs[r]], rbufs[r], semgs[r]).wait()

                for j in range(EB // 16):
                    w16 = wb[pl.ds(b * EB + j * 16, 16)]
                    for i in range(16):
                        wi = lax.gather(
                            w16, jnp.full((16, 1), i, jnp.int32),
                            lax.GatherDimensionNumbers(
                                offset_dims=(), collapsed_slice_dims=(0,),
                                start_index_map=(0,)),
                            (1,),
                            mode=lax.GatherScatterMode.PROMISE_IN_BOUNDS)
                        row = j * 16 + i
                        for k in range(GW // 16):
                            sl = pl.ds(k * 16, 16)
                            rbufs[r][row, sl] = rbufs[r][row, sl] * wi

                for j in range(EB // 16):
                    dvs[r][pl.ds(j * 16, 16)] = (
                        dstb[pl.ds(b * EB + j * 16, 16)])
                pltpu.async_copy(
                    rbufs[r], acc_s.at[dvs[r]], semss[r], add=True)
            return carry

        lax.fori_loop(0, nb // RD, body, 0)
        for s3 in ((nb - 2) % RD, (nb - 1) % RD):
            pltpu.make_async_copy(
                rbufs[s3], acc_s.at[dvs[s3]], semss[s3]).wait()
        plsc.subcore_barrier()

        @pl.when(sid <= NS - 2)
        def _():
            pltpu.sync_copy(acc_s.at[pl.ds(sid * 640, 640)],
                            acc_o.at[u, pl.ds(sid * 640, 640)])

        @pl.when(sid == NS - 1)
        def _():
            pltpu.sync_copy(
                acc_s.at[pl.ds((NS - 1) * 640, N - (NS - 1) * 640)],
                acc_o.at[u, pl.ds((NS - 1) * 640, N - (NS - 1) * 640)])

        plsc.subcore_barrier()
        return ccarry

    lax.fori_loop(0, P // NC, chunk_body, 0)


def _prop(gflat, src, dst, w):
    ept = E // NS
    return pl.kernel(
        _prop_body,
        out_type=jax.ShapeDtypeStruct((P, N, GW), _f32),
        mesh=_mesh,
        compiler_params=pltpu.CompilerParams(use_tc_tiling_on_sc=False),
        scratch_types=(
            [pltpu.VMEM((ept,), jnp.int32),
             pltpu.VMEM((ept,), jnp.int32),
             pltpu.VMEM((ept,), _f32)]
            + [pltpu.VMEM((EB,), jnp.int32) for _ in range(10)]
            + [pltpu.VMEM((EB, GW), _f32) for _ in range(5)]
            + [pltpu.VMEM_SHARED((N, GW), _f32)]
            + [pltpu.SemaphoreType.DMA for _ in range(10)]
        ),
    )(gflat, src, dst, w)


def _fin_body(acc, dinv, bcat, probs, out_o):
    u = pl.program_id(0)
    nb = pl.program_id(1)
    a = acc[0] * dinv[...] + bcat[...]
    z = jax.nn.sigmoid(a[:, 0:HD])
    t = jnp.tanh(a[:, HD:GW])
    pr = probs[0, u] * (1.0 / N)
    r = pr * jnp.sum((1.0 - z) * t, axis=0, keepdims=True)

    @pl.when((u == 0) & (nb == 0))
    def _():
        out_o[...] = r

    @pl.when((u > 0) | (nb > 0))
    def _():
        out_o[...] = out_o[...] + r


def _fin(acc, dinv, bcat, probs, nb):
    bn = N // nb
    return pl.pallas_call(
        _fin_body,
        grid=(P, nb),
        in_specs=[
            pl.BlockSpec((1, bn, GW), lambda u, n: (u, n, 0)),
            pl.BlockSpec((bn, 1), lambda u, n: (n, 0)),
            pl.BlockSpec((1, GW), lambda u, n: (0, 0)),
            pl.BlockSpec(memory_space=pltpu.SMEM),
        ],
        out_specs=pl.BlockSpec((1, HD), lambda u, n: (0, 0)),
        out_shape=jax.ShapeDtypeStruct((1, HD), _f32),
    )(acc, dinv, bcat, probs)


def kernel(X, edge_index, edge_weight, attention,
           W_conv_z, b_conv_z, W_lin_z, b_lin_z,
           W_conv_r, b_conv_r, W_lin_r, b_lin_r,
           W_conv_h, b_conv_h, W_lin_h, b_lin_h):
    ei = edge_index.astype(jnp.int32)
    src = ei[0]
    dst = ei[1]
    w = edge_weight.astype(_f32)

    wcat, bcat, probs = _wprep(
        W_conv_z, W_lin_z, b_conv_z.reshape(1, HD), b_lin_z.reshape(1, HD),
        W_conv_h, W_lin_h, b_conv_h.reshape(1, HD), b_lin_h.reshape(1, HD),
        attention.reshape(1, P))

    deg2 = _deg(dst, w)
    degt = deg2[:, :N].T

    xp = jnp.transpose(X, (2, 0, 1))
    gp, dinv = _gmat(xp, degt, wcat, nb=5)

    accf = _prop(gp.reshape(P * N, GW), src, dst, w)

    return _fin(accf, dinv, bcat, probs, nb=5)
